# baseline (speedup 1.0000x reference)
import jax
import jax.numpy as jnp
from jax import lax
from jax.experimental import pallas as pl
from jax.experimental.pallas import tpu as pltpu
from jax.experimental.pallas import tpu_sc as plsc

B = 16384
F = 26
FIELD_SIZE = 100000
TOTAL = F * FIELD_SIZE
LO = (TOTAL // 1024) * 1024
HI = TOTAL - LO
NC, NS = 2, 16
NW = NC * NS
R = B // NW
E = R * F
CHUNK = 512
NCHUNK = E // CHUNK
FIRE = 13
RC = R // 16


def _body(data_hbm, lo_hbm, hi_hbm, bias_hbm, out_hbm,
          data_v, idx_v, idxc_v, vals_v, out_v, hi_v, bias_v, sem):
    wid = lax.axis_index("s") * NC + lax.axis_index("c")

    pltpu.sync_copy(data_hbm.at[:, pl.ds(wid * R, R)], data_v)

    def field_phase(f, carry):
        def add_body(j, c2):
            sl = pl.ds(j * 16, 16)
            idx = data_v[f, sl] + f * FIELD_SIZE
            idxc_v[pl.ds(f * R + j * 16, 16)] = jnp.minimum(idx, LO - 1)
            return c2

        lax.fori_loop(0, RC, add_body, 0)
        sl = pl.ds(f * R, R)
        pltpu.async_copy(lo_hbm.at[idxc_v.at[sl]], vals_v.at[sl], sem.at[f])
        return carry

    lax.fori_loop(0, F - 1, field_phase, 0)

    def add25_body(j, carry):
        sl = pl.ds(j * 16, 16)
        idx = data_v[F - 1, sl] + (F - 1) * FIELD_SIZE
        idx_v[sl] = idx
        idxc_v[pl.ds((F - 1) * R + j * 16, 16)] = jnp.minimum(idx, LO - 1)
        return carry

    lax.fori_loop(0, RC, add25_body, 0)
    sl25 = pl.ds((F - 1) * R, R)
    h25 = pltpu.async_copy(lo_hbm.at[idxc_v.at[sl25]], vals_v.at[sl25],
                           sem.at[F - 1])

    pltpu.sync_copy(bias_hbm, bias_v)
    pltpu.sync_copy(hi_hbm, hi_v)
    bias_vec = bias_v[...]

    def init_body(c, carry):
        out_v[pl.ds(c * 16, 16)] = bias_vec
        return carry

    lax.fori_loop(0, RC, init_body, 0)

    def acc_phase(f, carry):
        sl = pl.ds(f * R, R)
        pltpu.make_async_copy(
            lo_hbm.at[idxc_v.at[sl]], vals_v.at[sl], sem.at[f]).wait()

        def accf_body(c, c2):
            slc = pl.ds(c * 16, 16)
            out_v[slc] = out_v[slc] + vals_v[pl.ds(f * R + c * 16, 16)]
            return c2

        lax.fori_loop(0, RC, accf_body, 0)
        return carry

    lax.fori_loop(0, F - 1, acc_phase, 0)

    def last_body(c, carry):
        rbase = c * 16
        sl = pl.ds((F - 1) * R + rbase, 16)
        v = vals_v[sl]
        io = idx_v[pl.ds(rbase, 16)]
        hv = plsc.load_gather(hi_v, [jnp.maximum(io - LO, 0)])
        acc = out_v[pl.ds(rbase, 16)] + jnp.where(io >= LO, hv, v)
        out_v[pl.ds(rbase, 16)] = 1.0 / (1.0 + jnp.exp(-acc))
        return carry

    h25.wait()
    lax.fori_loop(0, RC, last_body, 0)

    pltpu.sync_copy(out_v, out_hbm.at[pl.ds(wid * R, R)])


@jax.jit
def kernel(data, table, bias):
    mesh = plsc.VectorSubcoreMesh(
        core_axis_name="c", subcore_axis_name="s",
        num_cores=NC, num_subcores=NS)
    run = pl.kernel(
        _body,
        out_type=jax.ShapeDtypeStruct((B,), jnp.float32),
        mesh=mesh,
        compiler_params=pltpu.CompilerParams(needs_layout_passes=False),
        scratch_types=[
            pltpu.VMEM((F, R), jnp.int32),
            pltpu.VMEM((R,), jnp.int32),
            pltpu.VMEM((E,), jnp.int32),
            pltpu.VMEM((E,), jnp.float32),
            pltpu.VMEM((R,), jnp.float32),
            pltpu.VMEM((HI,), jnp.float32),
            pltpu.VMEM((16,), jnp.float32),
            pltpu.SemaphoreType.DMA((F,)),
        ],
    )
    bias16 = jnp.broadcast_to(bias.astype(jnp.float32), (16,))
    table_lo = lax.slice(table, (0, 0), (LO, 1)).reshape(LO)
    table_hi = lax.slice(table, (LO, 0), (TOTAL, 1)).reshape(HI)
    return run(data.T, table_lo, table_hi, bias16)


# speedup vs baseline: 1.0647x; 1.0647x over previous
import jax
import jax.numpy as jnp
from jax import lax
from jax.experimental import pallas as pl
from jax.experimental.pallas import tpu as pltpu
from jax.experimental.pallas import tpu_sc as plsc

B = 16384
F = 26
FIELD_SIZE = 100000
TOTAL = F * FIELD_SIZE
LO = (TOTAL // 1024) * 1024
HI = TOTAL - LO
NC, NS = 2, 16
NW = NC * NS
R = B // NW
E = R * F
CHUNK = 512
NCHUNK = E // CHUNK
FIRE = 13
RC = R // 16


def _body(data_hbm, lo_hbm, hi_hbm, bias_hbm, out_hbm,
          data_v, idx_v, idxc_v, vals_v, out_v, hi_v, bias_v, sem):
    wid = lax.axis_index("s") * NC + lax.axis_index("c")

    pltpu.sync_copy(data_hbm.at[:, pl.ds(wid * R, R)], data_v)
    pltpu.sync_copy(bias_hbm, bias_v)
    pltpu.sync_copy(hi_hbm, hi_v)

    def field_phase(f, carry):
        def add_body(j, c2):
            sl = pl.ds(j * 16, 16)
            idx = data_v[f, sl] + f * FIELD_SIZE
            idxc_v[pl.ds(f * R + j * 16, 16)] = jnp.minimum(idx, LO - 1)
            return c2

        lax.fori_loop(0, RC, add_body, 0)
        sl = pl.ds(f * R, R)
        pltpu.async_copy(lo_hbm.at[idxc_v.at[sl]], vals_v.at[sl], sem.at[f])
        return carry

    lax.fori_loop(0, F - 1, field_phase, 0)

    def add25_body(j, carry):
        sl = pl.ds(j * 16, 16)
        idx = data_v[F - 1, sl] + (F - 1) * FIELD_SIZE
        idx_v[sl] = idx
        idxc_v[pl.ds((F - 1) * R + j * 16, 16)] = jnp.minimum(idx, LO - 1)
        return carry

    lax.fori_loop(0, RC, add25_body, 0)
    sl25 = pl.ds((F - 1) * R, R)
    h25 = pltpu.async_copy(lo_hbm.at[idxc_v.at[sl25]], vals_v.at[sl25],
                           sem.at[F - 1])

    bias_vec = bias_v[...]

    def init_body(c, carry):
        out_v[pl.ds(c * 16, 16)] = bias_vec
        return carry

    lax.fori_loop(0, RC, init_body, 0)

    def acc_phase(f, carry):
        sl = pl.ds(f * R, R)
        pltpu.make_async_copy(
            lo_hbm.at[idxc_v.at[sl]], vals_v.at[sl], sem.at[f]).wait()

        def accf_body(c, c2):
            slc = pl.ds(c * 16, 16)
            out_v[slc] = out_v[slc] + vals_v[pl.ds(f * R + c * 16, 16)]
            return c2

        lax.fori_loop(0, RC, accf_body, 0)
        return carry

    lax.fori_loop(0, F - 1, acc_phase, 0)

    def last_body(c, carry):
        rbase = c * 16
        sl = pl.ds((F - 1) * R + rbase, 16)
        v = vals_v[sl]
        io = idx_v[pl.ds(rbase, 16)]
        hv = plsc.load_gather(hi_v, [jnp.maximum(io - LO, 0)])
        acc = out_v[pl.ds(rbase, 16)] + jnp.where(io >= LO, hv, v)
        out_v[pl.ds(rbase, 16)] = 1.0 / (1.0 + jnp.exp(-acc))
        return carry

    h25.wait()
    lax.fori_loop(0, RC, last_body, 0)

    pltpu.sync_copy(out_v, out_hbm.at[pl.ds(wid * R, R)])


@jax.jit
def kernel(data, table, bias):
    mesh = plsc.VectorSubcoreMesh(
        core_axis_name="c", subcore_axis_name="s",
        num_cores=NC, num_subcores=NS)
    run = pl.kernel(
        _body,
        out_type=jax.ShapeDtypeStruct((B,), jnp.float32),
        mesh=mesh,
        compiler_params=pltpu.CompilerParams(needs_layout_passes=False),
        scratch_types=[
            pltpu.VMEM((F, R), jnp.int32),
            pltpu.VMEM((R,), jnp.int32),
            pltpu.VMEM((E,), jnp.int32),
            pltpu.VMEM((E,), jnp.float32),
            pltpu.VMEM((R,), jnp.float32),
            pltpu.VMEM((HI,), jnp.float32),
            pltpu.VMEM((16,), jnp.float32),
            pltpu.SemaphoreType.DMA((F,)),
        ],
    )
    bias16 = jnp.broadcast_to(bias.astype(jnp.float32), (16,))
    table_lo = lax.slice(table, (0, 0), (LO, 1)).reshape(LO)
    table_hi = lax.slice(table, (LO, 0), (TOTAL, 1)).reshape(HI)
    return run(data.T, table_lo, table_hi, bias16)


# speedup vs baseline: 1.0683x; 1.0033x over previous
import jax
import jax.numpy as jnp
from jax import lax
from jax.experimental import pallas as pl
from jax.experimental.pallas import tpu as pltpu
from jax.experimental.pallas import tpu_sc as plsc

B = 16384
F = 26
FIELD_SIZE = 100000
TOTAL = F * FIELD_SIZE
LO = (TOTAL // 1024) * 1024
HI = TOTAL - LO
NC, NS = 2, 16
NW = NC * NS
R = B // NW
E = R * F
RC = R // 16


def _body(data_hbm, lo_hbm, hi_hbm, bias_hbm, out_hbm,
          data_v, idx_v, idxc_v, vals_v, out_v, hi_v, bias_v, sem):
    wid = lax.axis_index("s") * NC + lax.axis_index("c")

    pltpu.sync_copy(data_hbm.at[:, pl.ds(wid * R, R)], data_v)
    pltpu.sync_copy(bias_hbm, bias_v)
    pltpu.sync_copy(hi_hbm, hi_v)

    def field_phase(f, carry):
        def add_body(j, c2):
            sl = pl.ds(j * 16, 16)
            idx = data_v[f, sl] + f * FIELD_SIZE
            idxc_v[pl.ds(f * R + j * 16, 16)] = jnp.minimum(idx, LO - 1)
            return c2

        lax.fori_loop(0, RC, add_body, 0)
        sl = pl.ds(f * R, R)
        pltpu.async_copy(lo_hbm.at[idxc_v.at[sl]], vals_v.at[sl], sem.at[f])
        return carry

    lax.fori_loop(0, F - 1, field_phase, 0)

    def add25_body(j, carry):
        sl = pl.ds(j * 16, 16)
        idx = data_v[F - 1, sl] + (F - 1) * FIELD_SIZE
        idx_v[sl] = idx
        idxc_v[pl.ds((F - 1) * R + j * 16, 16)] = jnp.minimum(idx, LO - 1)
        return carry

    lax.fori_loop(0, RC, add25_body, 0)
    sl25 = pl.ds((F - 1) * R, R)
    h25 = pltpu.async_copy(lo_hbm.at[idxc_v.at[sl25]], vals_v.at[sl25],
                           sem.at[F - 1])

    bias_vec = bias_v[...]

    def init_body(c, carry):
        out_v[pl.ds(c * 16, 16)] = bias_vec
        return carry

    lax.fori_loop(0, RC, init_body, 0)

    def acc_phase(f, carry):
        sl = pl.ds(f * R, R)
        pltpu.make_async_copy(
            lo_hbm.at[idxc_v.at[sl]], vals_v.at[sl], sem.at[f]).wait()

        def accf_body(c, c2):
            slc = pl.ds(c * 16, 16)
            out_v[slc] = out_v[slc] + vals_v[pl.ds(f * R + c * 16, 16)]
            return c2

        lax.fori_loop(0, RC, accf_body, 0)
        return carry

    lax.fori_loop(0, F - 1, acc_phase, 0)

    def last_body(c, carry):
        rbase = c * 16
        sl = pl.ds((F - 1) * R + rbase, 16)
        v = vals_v[sl]
        io = idx_v[pl.ds(rbase, 16)]
        hv = plsc.load_gather(hi_v, [jnp.maximum(io - LO, 0)])
        acc = out_v[pl.ds(rbase, 16)] + jnp.where(io >= LO, hv, v)
        out_v[pl.ds(rbase, 16)] = 1.0 / (1.0 + jnp.exp(-acc))
        return carry

    h25.wait()
    lax.fori_loop(0, RC, last_body, 0)

    pltpu.sync_copy(out_v, out_hbm.at[pl.ds(wid * R, R)])


@jax.jit
def kernel(data, table, bias):
    mesh = plsc.VectorSubcoreMesh(
        core_axis_name="c", subcore_axis_name="s",
        num_cores=NC, num_subcores=NS)
    run = pl.kernel(
        _body,
        out_type=jax.ShapeDtypeStruct((B,), jnp.float32),
        mesh=mesh,
        compiler_params=pltpu.CompilerParams(needs_layout_passes=False),
        scratch_types=[
            pltpu.VMEM((F, R), jnp.int32),
            pltpu.VMEM((R,), jnp.int32),
            pltpu.VMEM((E,), jnp.int32),
            pltpu.VMEM((E,), jnp.float32),
            pltpu.VMEM((R,), jnp.float32),
            pltpu.VMEM((HI,), jnp.float32),
            pltpu.VMEM((16,), jnp.float32),
            pltpu.SemaphoreType.DMA((F,)),
        ],
    )
    bias16 = jnp.broadcast_to(bias.astype(jnp.float32), (16,))
    table_lo = lax.slice(table, (0, 0), (LO, 1)).reshape(LO)
    table_hi = lax.slice(table, (LO, 0), (TOTAL, 1)).reshape(HI)
    return run(data.T, table_lo, table_hi, bias16)
